# baseline (speedup 1.0000x reference)
import functools

import jax
import jax.numpy as jnp
import jax.scipy.linalg
from jax import lax
from jax.experimental import pallas as pl
from jax.experimental.pallas import tpu as pltpu
from jax.experimental.pallas import tpu_sc as plsc

N = 10000
NE = 320000
NW = 32
NT = 16
B = 100
BE = 12800
BN = 2000

_f32 = jnp.float32
_bf16 = jnp.bfloat16
_SC_PARAMS = pltpu.CompilerParams(use_tc_tiling_on_sc=False)


@functools.lru_cache(maxsize=None)
def _sc_gather_diff(nv, ne):
    ch = ne // NW
    j_n = ch // B
    b4 = B // 4
    mesh = plsc.VectorSubcoreMesh(core_axis_name="c", subcore_axis_name="s")

    assert j_n % 2 == 0

    @functools.partial(
        pl.kernel, mesh=mesh,
        out_type=jax.ShapeDtypeStruct((ne // 4, 128), _bf16),
        scratch_types=[
            pltpu.VMEM((j_n, B), jnp.int32),
            pltpu.VMEM((j_n, B), jnp.int32),
            pltpu.VMEM((B, 32), _bf16),
            pltpu.VMEM((B, 32), _bf16),
            pltpu.VMEM((B, 32), _bf16),
            pltpu.VMEM((B, 32), _bf16),
            pltpu.VMEM((b4, 128), _bf16),
            pltpu.VMEM((b4, 128), _bf16),
            pltpu.SemaphoreType.DMA,
            pltpu.SemaphoreType.DMA,
            pltpu.SemaphoreType.DMA,
            pltpu.SemaphoreType.DMA,
            pltpu.SemaphoreType.DMA,
            pltpu.SemaphoreType.DMA,
        ],
        compiler_params=_SC_PARAMS,
    )
    def k(tab, srcr, dstr, out, idx_s, idx_d, bs0, bd0, bs1, bd1, bo0, bo1,
          ss0, sd0, ss1, sd1, wo0, wo1):
        c = lax.axis_index("c")
        s = lax.axis_index("s")
        wid = c * NT + s
        pltpu.sync_copy(srcr.at[wid], idx_s)
        pltpu.sync_copy(dstr.at[wid], idx_d)
        base4 = wid * (ch // 4)
        sets = ((bs0, bd0, bo0, ss0, sd0, wo0), (bs1, bd1, bo1, ss1, sd1, wo1))

        def fire(j, p):
            bs, bd, _, ss, sd, _ = sets[p]
            pltpu.async_copy(tab.at[idx_s.at[j]], bs, ss)
            pltpu.async_copy(tab.at[idx_d.at[j]], bd, sd)

        fire(0, 0)

        def outer(jo, carry):
            for p in (0, 1):
                j = jo * 2 + p
                bs, bd, bo, ss, sd, wo = sets[p]

                @pl.when(j + 1 < j_n)
                def _():
                    fire(j + 1, 1 - p)

                pltpu.make_async_copy(tab.at[idx_s.at[j]], bs, ss).wait()
                pltpu.make_async_copy(tab.at[idx_d.at[j]], bd, sd).wait()

                @pl.when(jo >= 1)
                def _():
                    pltpu.make_async_copy(bo, out.at[pl.ds(0, b4)], wo).wait()

                for i in range(B):
                    bo[i // 4, pl.ds(32 * (i % 4), 32)] = (
                        bs[i, pl.ds(0, 32)] - bd[i, pl.ds(0, 32)])
                pltpu.async_copy(bo, out.at[pl.ds(base4 + j * b4, b4)], wo)
            return carry

        lax.fori_loop(0, j_n // 2, outer, 0)
        pltpu.make_async_copy(bo0, out.at[pl.ds(0, b4)], wo0).wait()
        pltpu.make_async_copy(bo1, out.at[pl.ds(0, b4)], wo1).wait()

    return k


@functools.lru_cache(maxsize=None)
def _sc_scatter_add(nv, ne):
    ch = ne // NW
    j_n = ch // B
    cb = 500
    ko_n = ch // cb
    spc = cb // B
    rt = nv // NT
    zr = 125
    nz = rt // zr
    assert ko_n % 2 == 0
    mesh = plsc.VectorSubcoreMesh(core_axis_name="c", subcore_axis_name="s")

    @functools.partial(
        pl.kernel, mesh=mesh,
        out_type=jax.ShapeDtypeStruct((2, nv, 32), _f32),
        scratch_types=[
            pltpu.VMEM_SHARED((nv, 32), _f32),
            pltpu.VMEM((zr, 32), _f32),
            pltpu.VMEM((j_n, B), jnp.int32),
            pltpu.VMEM((cb // 4, 128), _f32),
            pltpu.VMEM((cb // 4, 128), _f32),
            pltpu.VMEM((cb, 32), _f32),
            pltpu.SemaphoreType.DMA,
            pltpu.SemaphoreType.DMA,
            pltpu.SemaphoreType.DMA,
        ],
        compiler_params=_SC_PARAMS,
    )
    def k(eh4, dstr, out, acc, zbuf, idx_d, rb0, rb1, rs, sl0, sl1, sc):
        c = lax.axis_index("c")
        s = lax.axis_index("s")
        wid = c * NT + s
        zv = jnp.zeros((16,), _f32)
        for i in range(zr):
            for hh in (0, 16):
                zbuf[i, pl.ds(hh, 16)] = zv
        base_r = s * rt
        for r in range(nz):
            pltpu.sync_copy(zbuf, acc.at[pl.ds(base_r + r * zr, zr)])
        plsc.subcore_barrier()
        pltpu.sync_copy(dstr.at[wid], idx_d)
        sets = ((rb0, sl0), (rb1, sl1))
        ch4 = ch // 4
        cb4 = cb // 4

        def fire_load(io, p):
            pltpu.async_copy(eh4.at[pl.ds(wid * ch4 + io * cb4, cb4)],
                             sets[p][0], sets[p][1])

        fire_load(0, 0)

        def outer(ko, carry):
            for p in (0, 1):
                io = ko * 2 + p
                rb, sl = sets[p]

                @pl.when(io + 1 < ko_n)
                def _():
                    fire_load(io + 1, 1 - p)

                pltpu.make_async_copy(eh4.at[pl.ds(0, cb4)], rb, sl).wait()
                def repack(rg, carry2):
                    for rr in range(5):
                        r = rg * 5 + rr
                        for m in range(4):
                            for hh in (0, 16):
                                rs[4 * r + m, pl.ds(hh, 16)] = (
                                    rb[r, pl.ds(32 * m + hh, 16)])
                    return carry2

                lax.fori_loop(0, cb4 // 5, repack, 0)
                for q in range(spc):
                    pltpu.async_copy(rs.at[pl.ds(q * B, B)],
                                     acc.at[idx_d.at[io * spc + q]], sc,
                                     add=True)
                for q in range(spc):
                    pltpu.make_async_copy(
                        rs.at[pl.ds(0, B)], acc.at[idx_d.at[0]], sc).wait()
            return carry

        lax.fori_loop(0, ko_n // 2, outer, 0)
        plsc.subcore_barrier()
        pltpu.sync_copy(acc.at[pl.ds(base_r, rt)], out.at[c, pl.ds(base_r, rt)])

    return k


def _full(shape):
    return pl.BlockSpec(shape, lambda b: (0,) * len(shape))


def _blk(bs, w):
    return pl.BlockSpec((bs, w), lambda b: (b, 0))


def _relu(x):
    return jnp.maximum(x, 0.0)


def _dot(a, b):
    return jnp.dot(a, b, preferred_element_type=_f32)


def _t4(c):
    return jnp.concatenate([c, c, c, c], axis=1)


def _node_prep12(x_s, u_s, w):
    nv = x_s.shape[0]
    nb = nv // BN

    def body(x_ref, u_ref, wen1, ben1, wen2, ben2, wxt_e, wxb_e, wxt_n,
             wg1, bg1, wg2, bg2, xh0_ref, ptop_ref, qntop_ref, p1_ref, ucat_ref):
        xb = x_ref[...]
        h = _relu(_dot(xb, wen1[...]) + ben1[...])
        xh0 = _dot(h, wen2[...]) + ben2[...]
        ptop = _dot(xb, wxt_e[...])
        xh0_ref[...] = xh0
        ptop_ref[...] = ptop
        qntop_ref[...] = _dot(xb, wxt_n[...])
        p1_ref[...] = (ptop + _dot(xh0, wxb_e[...])).astype(_bf16)

        @pl.when(pl.program_id(0) == nb - 1)
        def _():
            u = u_ref[...]
            hg = _relu(_dot(u, wg1[...]) + bg1[...])
            uh0 = _dot(hg, wg2[...]) + bg2[...]
            ucat_ref[...] = jnp.concatenate([u, uh0], axis=1)

    g = u_s.shape[0]
    return pl.pallas_call(
        body,
        grid=(nb,),
        in_specs=[_blk(BN, 128), _full((g, 16))] + [_full(a.shape) for a in w],
        out_specs=[_blk(BN, 16), _blk(BN, 32), _blk(BN, 32), _blk(BN, 32),
                   _full((g, 32))],
        out_shape=[
            jax.ShapeDtypeStruct((nv, 16), _f32),
            jax.ShapeDtypeStruct((nv, 32), _f32),
            jax.ShapeDtypeStruct((nv, 32), _f32),
            jax.ShapeDtypeStruct((nv, 32), _bf16),
            jax.ShapeDtypeStruct((g, 32), _f32),
        ],
    )(x_s, u_s, *w)


def _node_prep3(xa4, xb4, uh2, w):
    n4 = N // 4

    def body(xa_ref, xb_ref, uh_ref, wen1, ben1, wen2, ben2, wxt_e, wxb_e,
             wxt_n, wg1, bg1, wg2, bg2, xh0_ref, ptop_ref, qntop_ref, p1_ref,
             ucat_ref):
        xd = xa_ref[...] - xb_ref[...]
        h = _relu(_dot(xd, wen1[...]) + ben1[...])
        xh0 = _dot(h, wen2[...]) + ben2[...]
        ptop = _dot(xd, wxt_e[...])
        xh0_ref[...] = xh0
        ptop_ref[...] = ptop
        qntop_ref[...] = _dot(xd, wxt_n[...])
        p1_ref[...] = (ptop + _dot(xh0, wxb_e[...])).astype(_bf16)
        ud = uh_ref[0:1, :] - uh_ref[1:2, :]
        hg = _relu(_dot(ud, wg1[...]) + bg1[...])
        uh0 = _dot(hg, wg2[...]) + bg2[...]
        ucat_ref[...] = jnp.concatenate([ud, uh0], axis=1)

    return pl.pallas_call(
        body,
        grid=(1,),
        in_specs=[_full((n4, 128)), _full((n4, 128)), _full((2, 16))]
        + [_full(a.shape) for a in w],
        out_specs=[_full((n4, 128))] * 4 + [_full((1, 32))],
        out_shape=[jax.ShapeDtypeStruct((n4, 128), _f32)] * 3
        + [jax.ShapeDtypeStruct((n4, 128), _bf16),
           jax.ShapeDtypeStruct((1, 32), _f32)],
    )(xa4, xb4, uh2, *w)


def _edge_prep(e4, w, diff_halves):
    ne4 = e4.shape[0] if not diff_halves else e4.shape[0] // 2
    bp = BE // 4
    nb = ne4 // bp

    def body(*refs):
        if diff_halves:
            eb = refs[0][...] - refs[1][...]
            rest = refs[2:]
        else:
            eb = refs[0][...]
            rest = refs[1:]
        we1, be1, we2, be2, wet, eh0_ref, qe_ref = rest
        h = _relu(_dot(eb, we1[...]) + be1[...])
        eh0_ref[...] = _dot(h, we2[...]) + be2[...]
        qe_ref[...] = _dot(eb, wet[...])

    if diff_halves:
        in_specs = [pl.BlockSpec((bp, 128), lambda b: (b, 0)),
                    pl.BlockSpec((bp, 128), lambda b: (b + nb, 0))]
        in_arrs = [e4, e4]
    else:
        in_specs = [_blk(bp, 128)]
        in_arrs = [e4]
    return pl.pallas_call(
        body,
        grid=(nb,),
        in_specs=in_specs + [_full(a.shape) for a in w],
        out_specs=[_blk(bp, 128), _blk(bp, 128)],
        out_shape=[jax.ShapeDtypeStruct((ne4, 128), _f32),
                   jax.ShapeDtypeStruct((ne4, 128), _f32)],
    )(*in_arrs, *w)


def _edge_pass(qe4, ehp4, d4, u_cat, w):
    ne4 = qe4.shape[0]
    g_n = u_cat.shape[0]
    bp = BE // 4
    nb = ne4 // bp
    per_g = nb // g_n

    def body(qe_ref, ehp_ref, d_ref, uc_ref, web, weu, b1, w2, b2t,
             eh_ref, esum_ref):
        b = pl.program_id(0)
        g = b // per_g
        mask = lax.broadcasted_iota(jnp.int32, (g_n, 1), 0) == g
        urow = jnp.sum(jnp.where(mask, uc_ref[...], 0.0), axis=0, keepdims=True)
        c = _t4(_dot(urow, weu[...]) + b1[...])
        h = _relu(qe_ref[...] + _dot(ehp_ref[...], web[...])
                  + d_ref[...].astype(_f32) + c)
        eh = _dot(h, w2[...]) + b2t[...]
        eh_ref[...] = eh

        @pl.when(b == 0)
        def _():
            esum_ref[...] = jnp.zeros_like(esum_ref)

        cs = jnp.sum(eh, axis=0, keepdims=True)
        cs16 = cs[:, 0:16] + cs[:, 32:48] + cs[:, 64:80] + cs[:, 96:112]
        esum_ref[...] += jnp.where(mask, cs16, 0.0)

    return pl.pallas_call(
        body,
        grid=(nb,),
        in_specs=[_blk(bp, 128)] * 3 + [_full((g_n, 32))]
        + [_full(a.shape) for a in w],
        out_specs=[_blk(bp, 128), _full((g_n, 16))],
        out_shape=[jax.ShapeDtypeStruct((ne4, 128), _f32),
                   jax.ShapeDtypeStruct((g_n, 16), _f32)],
    )(qe4, ehp4, d4, u_cat, *w)


def _node_pass(qntop4, xhp4, s0, s1, c0, c1, u_cat, esum, w, ptop4=None,
               wxb_e=None, wfin=None):
    nv4 = qntop4.shape[0]
    g_n = u_cat.shape[0]
    half = nv4 // g_n
    nv_g = half * 4
    out_p = ptop4 is not None
    out_fin = wfin is not None

    def body(*refs):
        qn_ref, xhp_ref, s0_ref, s1_ref, c0_ref, c1_ref, uc_ref = refs[:7]
        (wxb_n, wa, wu, b1, w2, b2t, wg1, bg1, wg2, bg2, esum_ref) = refs[7:18]
        pos = 18
        if out_p:
            ptop_ref = refs[pos]; wxb_ref = refs[pos + 1]; pos += 2
        if out_fin:
            wf1, bf1, wf2, bf2 = refs[pos:pos + 4]; pos += 4
        outs = refs[pos:]
        xh_ref, uh_ref, ucat_ref = outs[:3]
        rest = outs[3:]

        uc = uc_ref[...]
        cg = _dot(uc, wu[...]) + b1[...]
        if g_n == 2:
            rowi = lax.broadcasted_iota(jnp.int32, (nv4, 1), 0)
            cn = jnp.where(rowi < half, _t4(cg[0:1]), _t4(cg[1:2]))
        else:
            cn = _t4(cg)
        cnt = c0_ref[...] + c1_ref[...]
        agg = (s0_ref[...] + s1_ref[...]) / jnp.maximum(cnt, 1.0)
        pre = (qn_ref[...] + _dot(xhp_ref[...], wxb_n[...])
               + _dot(agg, wa[...]) + cn)
        xh = _dot(_relu(pre), w2[...]) + b2t[...]
        xh_ref[...] = xh
        if out_p:
            rest[0][...] = (ptop_ref[...] + _dot(xh, wxb_ref[...])).astype(_bf16)

        def colsum16(v):
            cs = jnp.sum(v, axis=0, keepdims=True)
            return cs[:, 0:16] + cs[:, 32:48] + cs[:, 64:80] + cs[:, 96:112]

        if g_n == 2:
            m0 = (rowi < half).astype(_f32)
            s_a = colsum16(xh * m0)
            s_b = colsum16(xh * (1.0 - m0))
            nsum = jnp.concatenate([s_a, s_b], axis=0)
        else:
            nsum = colsum16(xh)
        aggx = nsum * (1.0 / nv_g)
        aggeg = esum_ref[...] * (1.0 / NE)
        z = jnp.concatenate([aggx, aggeg, uc], axis=1)
        uh = _dot(_relu(_dot(z, wg1[...]) + bg1[...]), wg2[...]) + bg2[...]
        uh_ref[...] = uh
        ucat_ref[...] = jnp.concatenate([uc[:, 0:16], uh], axis=1)
        if out_fin:
            hf = _relu(_dot(uh, wf1[...]) + bf1[...])
            rest[-1][...] = _dot(hf, wf2[...]) + bf2[...]

    in_arrs = [qntop4, xhp4, s0, s1, c0, c1, u_cat] + list(w) + [esum]
    in_specs = ([_full((nv4, 128))] * 6 + [_full((g_n, 32))]
                + [_full(a.shape) for a in w] + [_full((g_n, 16))])
    if out_p:
        in_arrs += [ptop4, wxb_e]
        in_specs += [_full((nv4, 128)), _full(wxb_e.shape)]
    if out_fin:
        in_arrs += list(wfin)
        in_specs += [_full(a.shape) for a in wfin]
    out_specs = [_full((nv4, 128)), _full((g_n, 16)), _full((g_n, 32))]
    out_shape = [jax.ShapeDtypeStruct((nv4, 128), _f32),
                 jax.ShapeDtypeStruct((g_n, 16), _f32),
                 jax.ShapeDtypeStruct((g_n, 32), _f32)]
    if out_p:
        out_specs.append(_full((nv4, 128)))
        out_shape.append(jax.ShapeDtypeStruct((nv4, 128), _bf16))
    if out_fin:
        out_specs.append(_full((1, 2)))
        out_shape.append(jax.ShapeDtypeStruct((1, 2), _f32))
    return pl.pallas_call(
        body, grid=(1,), in_specs=in_specs, out_specs=out_specs,
        out_shape=out_shape,
    )(*in_arrs)


def _pad_rows(wt):
    return jnp.concatenate([wt, jnp.zeros((16, wt.shape[1]), _f32)], axis=0)


def _pad_cols(wt):
    return jnp.concatenate([wt, jnp.zeros((wt.shape[0], 16), _f32)], axis=1)


def _bd4(wt):
    return jax.scipy.linalg.block_diag(wt, wt, wt, wt)


def _t4h(bt):
    return jnp.tile(bt, (1, 4))


def _split_meta(rec, fx):
    pe, pn, pg = rec["edge"], rec["node"], rec["glob"]
    w1e, w1n = pe["W1"], pn["W1"]
    m = {
        "e_top": w1e[0:16], "e_bot": w1e[16:32],
        "e_xtop": w1e[32:32 + fx], "e_xbot": w1e[32 + fx:48 + fx],
        "e_u": w1e[48 + fx:80 + fx],
        "b1e": pe["b1"][None], "w2e": pe["W2"], "b2e": pe["b2"][None],
        "n_xtop": w1n[0:fx], "n_xbot": w1n[fx:fx + 16],
        "n_a": w1n[fx + 16:fx + 32], "n_u": w1n[fx + 32:fx + 64],
        "b1n": pn["b1"][None], "w2n": pn["W2"], "b2n": pn["b2"][None],
        "wg1": pg["W1"], "bg1": pg["b1"][None],
        "wg2": pg["W2"], "bg2": pg["b2"][None],
    }
    m["edge_w"] = [_bd4(_pad_rows(m["e_bot"])), m["e_u"], m["b1e"],
                   _bd4(_pad_cols(m["w2e"])), _t4h(_pad_cols(m["b2e"]))]
    m["node_w"] = [_bd4(_pad_rows(m["n_xbot"])), _bd4(_pad_rows(m["n_a"])),
                   m["n_u"], m["b1n"], _bd4(_pad_cols(m["w2n"])),
                   _t4h(_pad_cols(m["b2n"])), m["wg1"], m["bg1"], m["wg2"],
                   m["bg2"]]
    m["e_xbot_bd"] = _bd4(_pad_rows(m["e_xbot"]))
    return m


def _run_processing(nv, ne, srcr, dstr, c0, c1, prep_node, prep_edge, m,
                    final_w=None):
    xh0_4, ptop4, qntop4, p1, ucat1 = prep_node
    eh0_4, qe4 = prep_edge
    gather = _sc_gather_diff(nv, ne)
    scatter = _sc_scatter_add(nv, ne)

    def seg(eh4):
        sp = scatter(eh4, dstr)
        sp4 = jnp.reshape(sp, (2, nv // 4, 128))
        return sp4[0], sp4[1]

    d1 = gather(p1, srcr, dstr)
    eh1_4, esum1 = _edge_pass(qe4, eh0_4, d1, ucat1, m["edge_w"])
    s0, s1 = seg(eh1_4)
    xh1_4, uh1, ucat2, p2_4 = _node_pass(
        qntop4, xh0_4, s0, s1, c0, c1, ucat1, esum1, m["node_w"],
        ptop4=ptop4, wxb_e=m["e_xbot_bd"])

    d2 = gather(jnp.reshape(p2_4, (nv, 32)), srcr, dstr)
    eh2_4, esum2 = _edge_pass(qe4, eh1_4, d2, ucat2, m["edge_w"])
    s0, s1 = seg(eh2_4)
    res = _node_pass(qntop4, xh1_4, s0, s1, c0, c1, ucat2, esum2, m["node_w"],
                     wfin=final_w)
    if final_w is not None:
        return res[-1]
    xh2_4, uh2, _ = res
    return xh2_4, eh2_4, uh2


def kernel(x1, e1, u1, x2, e2, u2, edge_index, batch, params):
    src, dst = edge_index[0], edge_index[1]
    nv12, ne12 = 2 * N, 2 * NE
    x_s = jnp.concatenate([x1, x2], axis=0)
    ez = jnp.zeros((NE, 16), _f32)
    e4_s = jnp.reshape(
        jnp.concatenate([e1, ez, e2, ez], axis=1), (ne12 // 4, 128))
    u_s = jnp.concatenate([u1, u2], axis=0)
    src12 = jnp.concatenate([src, src + N]).reshape(NW, -1, B)
    dst12 = jnp.concatenate([dst, dst + N]).reshape(NW, -1, B)
    src3 = src.reshape(NW, -1, B)
    dst3 = dst.reshape(NW, -1, B)

    ones = jnp.reshape(
        jnp.concatenate([jnp.ones((NE, 16), _f32),
                         jnp.zeros((NE, 16), _f32)], axis=1),
        (NE // 4, 128))
    counts = _sc_scatter_add(N, NE)(ones, dst3)
    c0_3 = jnp.reshape(counts[0], (N // 4, 128))
    c1_3 = jnp.reshape(counts[1], (N // 4, 128))
    c0 = jnp.concatenate([c0_3, c0_3], axis=0)
    c1 = jnp.concatenate([c1_3, c1_3], axis=0)

    m12 = _split_meta(params["rec"], 128)
    m3 = _split_meta(params["recf"], 16)
    enc, encf, fin = params["enc"], params["encf"], params["final"]

    pn12 = _node_prep12(
        x_s, u_s,
        [enc["node"]["W1"], enc["node"]["b1"][None], enc["node"]["W2"],
         enc["node"]["b2"][None], m12["e_xtop"], m12["e_xbot"], m12["n_xtop"],
         enc["glob"]["W1"], enc["glob"]["b1"][None], enc["glob"]["W2"],
         enc["glob"]["b2"][None]])
    xh0, ptop, qntop, p1, ucat1 = pn12
    xh0_4 = jnp.reshape(
        jnp.concatenate([xh0, jnp.zeros((nv12, 16), _f32)], axis=1),
        (nv12 // 4, 128))
    prep_node12 = (xh0_4, jnp.reshape(ptop, (nv12 // 4, 128)),
                   jnp.reshape(qntop, (nv12 // 4, 128)), p1, ucat1)

    prep_edge12 = _edge_prep(
        e4_s,
        [_bd4(_pad_rows(enc["edge"]["W1"])), _t4h(enc["edge"]["b1"][None]),
         _bd4(_pad_cols(enc["edge"]["W2"])),
         _t4h(_pad_cols(enc["edge"]["b2"][None])),
         _bd4(_pad_rows(m12["e_top"]))],
        diff_halves=False)

    xh2_4, eh2_4, uh2 = _run_processing(
        nv12, ne12, src12, dst12, c0, c1, prep_node12, prep_edge12, m12)

    pn3 = _node_prep3(
        xh2_4[:N // 4], xh2_4[N // 4:], uh2,
        [_bd4(_pad_rows(encf["node"]["W1"])), _t4h(encf["node"]["b1"][None]),
         _bd4(_pad_cols(encf["node"]["W2"])),
         _t4h(_pad_cols(encf["node"]["b2"][None])),
         _bd4(_pad_rows(m3["e_xtop"])), _bd4(_pad_rows(m3["e_xbot"])),
         _bd4(_pad_rows(m3["n_xtop"])),
         encf["glob"]["W1"], encf["glob"]["b1"][None], encf["glob"]["W2"],
         encf["glob"]["b2"][None]])
    xh0d_4, ptopd4, qntopd4, p1d_4, ucat1d = pn3
    prep_node3 = (xh0d_4, ptopd4, qntopd4, jnp.reshape(p1d_4, (N, 32)), ucat1d)

    prep_edge3 = _edge_prep(
        eh2_4,
        [_bd4(_pad_rows(encf["edge"]["W1"])), _t4h(encf["edge"]["b1"][None]),
         _bd4(_pad_cols(encf["edge"]["W2"])),
         _t4h(_pad_cols(encf["edge"]["b2"][None])),
         _bd4(_pad_rows(m3["e_top"]))],
        diff_halves=True)

    final_w = [fin["W1"], fin["b1"][None], fin["W2"], fin["b2"][None]]
    return _run_processing(
        N, NE, src3, dst3, c0_3, c1_3, prep_node3, prep_edge3, m3,
        final_w=final_w)


# speedup vs baseline: 1.3380x; 1.3380x over previous
import functools

import jax
import jax.numpy as jnp
import jax.scipy.linalg
from jax import lax
from jax.experimental import pallas as pl
from jax.experimental.pallas import tpu as pltpu
from jax.experimental.pallas import tpu_sc as plsc

N = 10000
NE = 320000
NW = 32
NT = 16
B = 100
BE = 12800
BN = 2000

_f32 = jnp.float32
_bf16 = jnp.bfloat16
_SC_PARAMS = pltpu.CompilerParams(use_tc_tiling_on_sc=False)


@functools.lru_cache(maxsize=None)
def _sc_gather_diff(nv, ne):
    ch = ne // NW
    j_n = ch // B
    b4 = B // 4
    mesh = plsc.VectorSubcoreMesh(core_axis_name="c", subcore_axis_name="s")

    assert j_n % 2 == 0

    @functools.partial(
        pl.kernel, mesh=mesh,
        out_type=jax.ShapeDtypeStruct((ne // 4, 128), _f32),
        scratch_types=[
            pltpu.VMEM((j_n, B), jnp.int32),
            pltpu.VMEM((j_n, B), jnp.int32),
            pltpu.VMEM((B, 32), _f32),
            pltpu.VMEM((B, 32), _f32),
            pltpu.VMEM((B, 32), _f32),
            pltpu.VMEM((B, 32), _f32),
            pltpu.VMEM((b4, 128), _f32),
            pltpu.VMEM((b4, 128), _f32),
            pltpu.SemaphoreType.DMA,
            pltpu.SemaphoreType.DMA,
            pltpu.SemaphoreType.DMA,
            pltpu.SemaphoreType.DMA,
            pltpu.SemaphoreType.DMA,
            pltpu.SemaphoreType.DMA,
        ],
        compiler_params=_SC_PARAMS,
    )
    def k(tab, srcr, dstr, out, idx_s, idx_d, bs0, bd0, bs1, bd1, bo0, bo1,
          ss0, sd0, ss1, sd1, wo0, wo1):
        c = lax.axis_index("c")
        s = lax.axis_index("s")
        wid = c * NT + s
        pltpu.sync_copy(srcr.at[wid], idx_s)
        pltpu.sync_copy(dstr.at[wid], idx_d)
        base4 = wid * (ch // 4)
        sets = ((bs0, bd0, bo0, ss0, sd0, wo0), (bs1, bd1, bo1, ss1, sd1, wo1))

        def fire(j, p):
            bs, bd, _, ss, sd, _ = sets[p]
            pltpu.async_copy(tab.at[idx_s.at[j]], bs, ss)
            pltpu.async_copy(tab.at[idx_d.at[j]], bd, sd)

        fire(0, 0)

        def outer(jo, carry):
            for p in (0, 1):
                j = jo * 2 + p
                bs, bd, bo, ss, sd, wo = sets[p]

                @pl.when(j + 1 < j_n)
                def _():
                    fire(j + 1, 1 - p)

                pltpu.make_async_copy(tab.at[idx_s.at[j]], bs, ss).wait()
                pltpu.make_async_copy(tab.at[idx_d.at[j]], bd, sd).wait()

                @pl.when(jo >= 1)
                def _():
                    pltpu.make_async_copy(bo, out.at[pl.ds(0, b4)], wo).wait()

                for i in range(B):
                    for hh in (0, 16):
                        bo[i // 4, pl.ds(32 * (i % 4) + hh, 16)] = (
                            bs[i, pl.ds(hh, 16)] - bd[i, pl.ds(hh, 16)])
                pltpu.async_copy(bo, out.at[pl.ds(base4 + j * b4, b4)], wo)
            return carry

        lax.fori_loop(0, j_n // 2, outer, 0)
        pltpu.make_async_copy(bo0, out.at[pl.ds(0, b4)], wo0).wait()
        pltpu.make_async_copy(bo1, out.at[pl.ds(0, b4)], wo1).wait()

    return k


@functools.lru_cache(maxsize=None)
def _sc_scatter_add(nv, ne):
    ch = ne // NW
    j_n = ch // B
    cb = 500
    ko_n = ch // cb
    spc = cb // B
    rt = nv // NT
    zr = 125
    nz = rt // zr
    assert ko_n % 2 == 0
    mesh = plsc.VectorSubcoreMesh(core_axis_name="c", subcore_axis_name="s")

    @functools.partial(
        pl.kernel, mesh=mesh,
        out_type=jax.ShapeDtypeStruct((2, nv, 32), _f32),
        scratch_types=[
            pltpu.VMEM_SHARED((nv, 32), _f32),
            pltpu.VMEM((zr, 32), _f32),
            pltpu.VMEM((j_n, B), jnp.int32),
            pltpu.VMEM((cb // 4, 128), _f32),
            pltpu.VMEM((cb // 4, 128), _f32),
            pltpu.VMEM((cb, 32), _f32),
            pltpu.SemaphoreType.DMA,
            pltpu.SemaphoreType.DMA,
            pltpu.SemaphoreType.DMA,
        ],
        compiler_params=_SC_PARAMS,
    )
    def k(eh4, dstr, out, acc, zbuf, idx_d, rb0, rb1, rs, sl0, sl1, sc):
        c = lax.axis_index("c")
        s = lax.axis_index("s")
        wid = c * NT + s
        zv = jnp.zeros((16,), _f32)
        for i in range(zr):
            for hh in (0, 16):
                zbuf[i, pl.ds(hh, 16)] = zv
        base_r = s * rt
        for r in range(nz):
            pltpu.sync_copy(zbuf, acc.at[pl.ds(base_r + r * zr, zr)])
        plsc.subcore_barrier()
        pltpu.sync_copy(dstr.at[wid], idx_d)
        sets = ((rb0, sl0), (rb1, sl1))
        ch4 = ch // 4
        cb4 = cb // 4

        def fire_load(io, p):
            pltpu.async_copy(eh4.at[pl.ds(wid * ch4 + io * cb4, cb4)],
                             sets[p][0], sets[p][1])

        fire_load(0, 0)

        def outer(ko, carry):
            for p in (0, 1):
                io = ko * 2 + p
                rb, sl = sets[p]

                @pl.when(io + 1 < ko_n)
                def _():
                    fire_load(io + 1, 1 - p)

                pltpu.make_async_copy(eh4.at[pl.ds(0, cb4)], rb, sl).wait()
                def repack(rg, carry2):
                    for rr in range(5):
                        r = rg * 5 + rr
                        for m in range(4):
                            for hh in (0, 16):
                                rs[4 * r + m, pl.ds(hh, 16)] = (
                                    rb[r, pl.ds(32 * m + hh, 16)])
                    return carry2

                lax.fori_loop(0, cb4 // 5, repack, 0)
                for q in range(spc):
                    pltpu.async_copy(rs.at[pl.ds(q * B, B)],
                                     acc.at[idx_d.at[io * spc + q]], sc,
                                     add=True)
                for q in range(spc):
                    pltpu.make_async_copy(
                        rs.at[pl.ds(0, B)], acc.at[idx_d.at[0]], sc).wait()
            return carry

        lax.fori_loop(0, ko_n // 2, outer, 0)
        plsc.subcore_barrier()
        pltpu.sync_copy(acc.at[pl.ds(base_r, rt)], out.at[c, pl.ds(base_r, rt)])

    return k


def _full(shape):
    return pl.BlockSpec(shape, lambda b: (0,) * len(shape))


def _blk(bs, w):
    return pl.BlockSpec((bs, w), lambda b: (b, 0))


def _relu(x):
    return jnp.maximum(x, 0.0)


def _dot(a, b):
    return jnp.dot(a, b, preferred_element_type=_f32)


def _t4(c):
    return jnp.concatenate([c, c, c, c], axis=1)


def _node_prep12(x_s, u_s, w):
    nv = x_s.shape[0]
    nb = nv // BN

    def body(x_ref, u_ref, wen1, ben1, wen2, ben2, wxt_e, wxb_e, wxt_n,
             wg1, bg1, wg2, bg2, xh0_ref, ptop_ref, qntop_ref, p1_ref, ucat_ref):
        xb = x_ref[...]
        h = _relu(_dot(xb, wen1[...]) + ben1[...])
        xh0 = _dot(h, wen2[...]) + ben2[...]
        ptop = _dot(xb, wxt_e[...])
        xh0_ref[...] = xh0
        ptop_ref[...] = ptop
        qntop_ref[...] = _dot(xb, wxt_n[...])
        p1_ref[...] = ptop + _dot(xh0, wxb_e[...])

        @pl.when(pl.program_id(0) == nb - 1)
        def _():
            u = u_ref[...]
            hg = _relu(_dot(u, wg1[...]) + bg1[...])
            uh0 = _dot(hg, wg2[...]) + bg2[...]
            ucat_ref[...] = jnp.concatenate([u, uh0], axis=1)

    g = u_s.shape[0]
    return pl.pallas_call(
        body,
        grid=(nb,),
        in_specs=[_blk(BN, 128), _full((g, 16))] + [_full(a.shape) for a in w],
        out_specs=[_blk(BN, 16), _blk(BN, 32), _blk(BN, 32), _blk(BN, 32),
                   _full((g, 32))],
        out_shape=[
            jax.ShapeDtypeStruct((nv, 16), _f32),
            jax.ShapeDtypeStruct((nv, 32), _f32),
            jax.ShapeDtypeStruct((nv, 32), _f32),
            jax.ShapeDtypeStruct((nv, 32), _f32),
            jax.ShapeDtypeStruct((g, 32), _f32),
        ],
    )(x_s, u_s, *w)


def _node_prep3(xa4, xb4, uh2, w):
    n4 = N // 4

    def body(xa_ref, xb_ref, uh_ref, wen1, ben1, wen2, ben2, wxt_e, wxb_e,
             wxt_n, wg1, bg1, wg2, bg2, xh0_ref, ptop_ref, qntop_ref, p1_ref,
             ucat_ref):
        xd = xa_ref[...] - xb_ref[...]
        h = _relu(_dot(xd, wen1[...]) + ben1[...])
        xh0 = _dot(h, wen2[...]) + ben2[...]
        ptop = _dot(xd, wxt_e[...])
        xh0_ref[...] = xh0
        ptop_ref[...] = ptop
        qntop_ref[...] = _dot(xd, wxt_n[...])
        p1_ref[...] = ptop + _dot(xh0, wxb_e[...])
        ud = uh_ref[0:1, :] - uh_ref[1:2, :]
        hg = _relu(_dot(ud, wg1[...]) + bg1[...])
        uh0 = _dot(hg, wg2[...]) + bg2[...]
        ucat_ref[...] = jnp.concatenate([ud, uh0], axis=1)

    return pl.pallas_call(
        body,
        grid=(1,),
        in_specs=[_full((n4, 128)), _full((n4, 128)), _full((2, 16))]
        + [_full(a.shape) for a in w],
        out_specs=[_full((n4, 128))] * 4 + [_full((1, 32))],
        out_shape=[jax.ShapeDtypeStruct((n4, 128), _f32)] * 4
        + [jax.ShapeDtypeStruct((1, 32), _f32)],
    )(xa4, xb4, uh2, *w)


def _edge_prep(e4, w, diff_halves):
    ne4 = e4.shape[0] if not diff_halves else e4.shape[0] // 2
    bp = BE // 4
    nb = ne4 // bp

    def body(*refs):
        if diff_halves:
            eb = refs[0][...] - refs[1][...]
            rest = refs[2:]
        else:
            eb = refs[0][...]
            rest = refs[1:]
        we1, be1, we2, be2, wet, eh0_ref, qe_ref = rest
        h = _relu(_dot(eb, we1[...]) + be1[...])
        eh0_ref[...] = _dot(h, we2[...]) + be2[...]
        qe_ref[...] = _dot(eb, wet[...])

    if diff_halves:
        in_specs = [pl.BlockSpec((bp, 128), lambda b: (b, 0)),
                    pl.BlockSpec((bp, 128), lambda b: (b + nb, 0))]
        in_arrs = [e4, e4]
    else:
        in_specs = [_blk(bp, 128)]
        in_arrs = [e4]
    return pl.pallas_call(
        body,
        grid=(nb,),
        in_specs=in_specs + [_full(a.shape) for a in w],
        out_specs=[_blk(bp, 128), _blk(bp, 128)],
        out_shape=[jax.ShapeDtypeStruct((ne4, 128), _f32),
                   jax.ShapeDtypeStruct((ne4, 128), _f32)],
    )(*in_arrs, *w)


def _edge_pass(qe4, ehp4, d4, u_cat, w):
    ne4 = qe4.shape[0]
    g_n = u_cat.shape[0]
    bp = BE // 4
    nb = ne4 // bp
    per_g = nb // g_n

    def body(qe_ref, ehp_ref, d_ref, uc_ref, web, weu, b1, w2, b2t,
             eh_ref, esum_ref):
        b = pl.program_id(0)
        g = b // per_g
        mask = lax.broadcasted_iota(jnp.int32, (g_n, 1), 0) == g
        urow = jnp.sum(jnp.where(mask, uc_ref[...], 0.0), axis=0, keepdims=True)
        c = _t4(_dot(urow, weu[...]) + b1[...])
        h = _relu(qe_ref[...] + _dot(ehp_ref[...], web[...]) + d_ref[...] + c)
        eh = _dot(h, w2[...]) + b2t[...]
        eh_ref[...] = eh

        @pl.when(b == 0)
        def _():
            esum_ref[...] = jnp.zeros_like(esum_ref)

        cs = jnp.sum(eh, axis=0, keepdims=True)
        cs16 = cs[:, 0:16] + cs[:, 32:48] + cs[:, 64:80] + cs[:, 96:112]
        esum_ref[...] += jnp.where(mask, cs16, 0.0)

    return pl.pallas_call(
        body,
        grid=(nb,),
        in_specs=[_blk(bp, 128)] * 3 + [_full((g_n, 32))]
        + [_full(a.shape) for a in w],
        out_specs=[_blk(bp, 128), _full((g_n, 16))],
        out_shape=[jax.ShapeDtypeStruct((ne4, 128), _f32),
                   jax.ShapeDtypeStruct((g_n, 16), _f32)],
    )(qe4, ehp4, d4, u_cat, *w)


def _node_pass(qntop4, xhp4, s0, s1, u_cat, esum, w, ptop4=None,
               wxb_e=None, wfin=None):
    nv4 = qntop4.shape[0]
    g_n = u_cat.shape[0]
    half = nv4 // g_n
    nv_g = half * 4
    out_p = ptop4 is not None
    out_fin = wfin is not None

    def body(*refs):
        qn_ref, xhp_ref, s0_ref, s1_ref, uc_ref = refs[:5]
        (wxb_n, wa, wu, b1, w2, b2t, wg1, bg1, wg2, bg2, esum_ref) = refs[5:16]
        pos = 16
        if out_p:
            ptop_ref = refs[pos]; wxb_ref = refs[pos + 1]; pos += 2
        if out_fin:
            wf1, bf1, wf2, bf2 = refs[pos:pos + 4]; pos += 4
        outs = refs[pos:]
        xh_ref, uh_ref, ucat_ref = outs[:3]
        rest = outs[3:]

        uc = uc_ref[...]
        cg = _dot(uc, wu[...]) + b1[...]
        if g_n == 2:
            rowi = lax.broadcasted_iota(jnp.int32, (nv4, 1), 0)
            cn = jnp.where(rowi < half, _t4(cg[0:1]), _t4(cg[1:2]))
        else:
            cn = _t4(cg)
        sboth = s0_ref[...] + s1_ref[...]
        shifted = jnp.concatenate(
            [jnp.broadcast_to(sboth[:, 32 * q + 16:32 * q + 17], (nv4, 32))
             for q in range(4)], axis=1)
        agg = sboth / jnp.maximum(shifted, 1.0)
        pre = (qn_ref[...] + _dot(xhp_ref[...], wxb_n[...])
               + _dot(agg, wa[...]) + cn)
        xh = _dot(_relu(pre), w2[...]) + b2t[...]
        xh_ref[...] = xh
        if out_p:
            rest[0][...] = ptop_ref[...] + _dot(xh, wxb_ref[...])

        def colsum16(v):
            cs = jnp.sum(v, axis=0, keepdims=True)
            return cs[:, 0:16] + cs[:, 32:48] + cs[:, 64:80] + cs[:, 96:112]

        if g_n == 2:
            m0 = (rowi < half).astype(_f32)
            s_a = colsum16(xh * m0)
            s_b = colsum16(xh * (1.0 - m0))
            nsum = jnp.concatenate([s_a, s_b], axis=0)
        else:
            nsum = colsum16(xh)
        aggx = nsum * (1.0 / nv_g)
        aggeg = esum_ref[...] * (1.0 / NE)
        z = jnp.concatenate([aggx, aggeg, uc], axis=1)
        uh = _dot(_relu(_dot(z, wg1[...]) + bg1[...]), wg2[...]) + bg2[...]
        uh_ref[...] = uh
        ucat_ref[...] = jnp.concatenate([uc[:, 0:16], uh], axis=1)
        if out_fin:
            hf = _relu(_dot(uh, wf1[...]) + bf1[...])
            rest[-1][...] = _dot(hf, wf2[...]) + bf2[...]

    in_arrs = [qntop4, xhp4, s0, s1, u_cat] + list(w) + [esum]
    in_specs = ([_full((nv4, 128))] * 4 + [_full((g_n, 32))]
                + [_full(a.shape) for a in w] + [_full((g_n, 16))])
    if out_p:
        in_arrs += [ptop4, wxb_e]
        in_specs += [_full((nv4, 128)), _full(wxb_e.shape)]
    if out_fin:
        in_arrs += list(wfin)
        in_specs += [_full(a.shape) for a in wfin]
    out_specs = [_full((nv4, 128)), _full((g_n, 16)), _full((g_n, 32))]
    out_shape = [jax.ShapeDtypeStruct((nv4, 128), _f32),
                 jax.ShapeDtypeStruct((g_n, 16), _f32),
                 jax.ShapeDtypeStruct((g_n, 32), _f32)]
    if out_p:
        out_specs.append(_full((nv4, 128)))
        out_shape.append(jax.ShapeDtypeStruct((nv4, 128), _f32))
    if out_fin:
        out_specs.append(_full((1, 2)))
        out_shape.append(jax.ShapeDtypeStruct((1, 2), _f32))
    return pl.pallas_call(
        body, grid=(1,), in_specs=in_specs, out_specs=out_specs,
        out_shape=out_shape,
    )(*in_arrs)


def _pad_rows(wt):
    return jnp.concatenate([wt, jnp.zeros((16, wt.shape[1]), _f32)], axis=0)


def _pad_cols(wt):
    return jnp.concatenate([wt, jnp.zeros((wt.shape[0], 16), _f32)], axis=1)


def _bd4(wt):
    return jax.scipy.linalg.block_diag(wt, wt, wt, wt)


def _t4h(bt):
    return jnp.tile(bt, (1, 4))


def _split_meta(rec, fx):
    pe, pn, pg = rec["edge"], rec["node"], rec["glob"]
    w1e, w1n = pe["W1"], pn["W1"]
    m = {
        "e_top": w1e[0:16], "e_bot": w1e[16:32],
        "e_xtop": w1e[32:32 + fx], "e_xbot": w1e[32 + fx:48 + fx],
        "e_u": w1e[48 + fx:80 + fx],
        "b1e": pe["b1"][None], "w2e": pe["W2"], "b2e": pe["b2"][None],
        "n_xtop": w1n[0:fx], "n_xbot": w1n[fx:fx + 16],
        "n_a": w1n[fx + 16:fx + 32], "n_u": w1n[fx + 32:fx + 64],
        "b1n": pn["b1"][None], "w2n": pn["W2"], "b2n": pn["b2"][None],
        "wg1": pg["W1"], "bg1": pg["b1"][None],
        "wg2": pg["W2"], "bg2": pg["b2"][None],
    }
    b2cnt = jnp.concatenate(
        [m["b2e"], jnp.ones((1, 1), _f32), jnp.zeros((1, 15), _f32)], axis=1)
    m["edge_w"] = [_bd4(_pad_rows(m["e_bot"])), m["e_u"], m["b1e"],
                   _bd4(_pad_cols(m["w2e"])), _t4h(b2cnt)]
    m["node_w"] = [_bd4(_pad_rows(m["n_xbot"])), _bd4(_pad_rows(m["n_a"])),
                   m["n_u"], m["b1n"], _bd4(_pad_cols(m["w2n"])),
                   _t4h(_pad_cols(m["b2n"])), m["wg1"], m["bg1"], m["wg2"],
                   m["bg2"]]
    m["e_xbot_bd"] = _bd4(_pad_rows(m["e_xbot"]))
    return m


def _run_processing(nv, ne, srcr, dstr, prep_node, prep_edge, m,
                    final_w=None):
    xh0_4, ptop4, qntop4, p1, ucat1 = prep_node
    eh0_4, qe4 = prep_edge
    gather = _sc_gather_diff(nv, ne)
    scatter = _sc_scatter_add(nv, ne)

    def seg(eh4):
        sp = scatter(eh4, dstr)
        sp4 = jnp.reshape(sp, (2, nv // 4, 128))
        return sp4[0], sp4[1]

    d1 = gather(p1, srcr, dstr)
    eh1_4, esum1 = _edge_pass(qe4, eh0_4, d1, ucat1, m["edge_w"])
    s0, s1 = seg(eh1_4)
    xh1_4, uh1, ucat2, p2_4 = _node_pass(
        qntop4, xh0_4, s0, s1, ucat1, esum1, m["node_w"],
        ptop4=ptop4, wxb_e=m["e_xbot_bd"])

    d2 = gather(jnp.reshape(p2_4, (nv, 32)), srcr, dstr)
    eh2_4, esum2 = _edge_pass(qe4, eh1_4, d2, ucat2, m["edge_w"])
    s0, s1 = seg(eh2_4)
    res = _node_pass(qntop4, xh1_4, s0, s1, ucat2, esum2, m["node_w"],
                     wfin=final_w)
    if final_w is not None:
        return res[-1]
    xh2_4, uh2, _ = res
    return xh2_4, eh2_4, uh2


def kernel(x1, e1, u1, x2, e2, u2, edge_index, batch, params):
    src, dst = edge_index[0], edge_index[1]
    nv12, ne12 = 2 * N, 2 * NE
    x_s = jnp.concatenate([x1, x2], axis=0)
    ez = jnp.zeros((NE, 16), _f32)
    e4_s = jnp.reshape(
        jnp.concatenate([e1, ez, e2, ez], axis=1), (ne12 // 4, 128))
    u_s = jnp.concatenate([u1, u2], axis=0)
    src12 = jnp.concatenate([src, src + N]).reshape(NW, -1, B)
    dst12 = jnp.concatenate([dst, dst + N]).reshape(NW, -1, B)
    src3 = src.reshape(NW, -1, B)
    dst3 = dst.reshape(NW, -1, B)

    m12 = _split_meta(params["rec"], 128)
    m3 = _split_meta(params["recf"], 16)
    enc, encf, fin = params["enc"], params["encf"], params["final"]

    pn12 = _node_prep12(
        x_s, u_s,
        [enc["node"]["W1"], enc["node"]["b1"][None], enc["node"]["W2"],
         enc["node"]["b2"][None], m12["e_xtop"], m12["e_xbot"], m12["n_xtop"],
         enc["glob"]["W1"], enc["glob"]["b1"][None], enc["glob"]["W2"],
         enc["glob"]["b2"][None]])
    xh0, ptop, qntop, p1, ucat1 = pn12
    xh0_4 = jnp.reshape(
        jnp.concatenate([xh0, jnp.zeros((nv12, 16), _f32)], axis=1),
        (nv12 // 4, 128))
    prep_node12 = (xh0_4, jnp.reshape(ptop, (nv12 // 4, 128)),
                   jnp.reshape(qntop, (nv12 // 4, 128)), p1, ucat1)

    prep_edge12 = _edge_prep(
        e4_s,
        [_bd4(_pad_rows(enc["edge"]["W1"])), _t4h(enc["edge"]["b1"][None]),
         _bd4(_pad_cols(enc["edge"]["W2"])),
         _t4h(_pad_cols(enc["edge"]["b2"][None])),
         _bd4(_pad_rows(m12["e_top"]))],
        diff_halves=False)

    xh2_4, eh2_4, uh2 = _run_processing(
        nv12, ne12, src12, dst12, prep_node12, prep_edge12, m12)

    pn3 = _node_prep3(
        xh2_4[:N // 4], xh2_4[N // 4:], uh2,
        [_bd4(_pad_rows(encf["node"]["W1"])), _t4h(encf["node"]["b1"][None]),
         _bd4(_pad_cols(encf["node"]["W2"])),
         _t4h(_pad_cols(encf["node"]["b2"][None])),
         _bd4(_pad_rows(m3["e_xtop"])), _bd4(_pad_rows(m3["e_xbot"])),
         _bd4(_pad_rows(m3["n_xtop"])),
         encf["glob"]["W1"], encf["glob"]["b1"][None], encf["glob"]["W2"],
         encf["glob"]["b2"][None]])
    xh0d_4, ptopd4, qntopd4, p1d_4, ucat1d = pn3
    prep_node3 = (xh0d_4, ptopd4, qntopd4, jnp.reshape(p1d_4, (N, 32)), ucat1d)

    prep_edge3 = _edge_prep(
        eh2_4,
        [_bd4(_pad_rows(encf["edge"]["W1"])), _t4h(encf["edge"]["b1"][None]),
         _bd4(_pad_cols(encf["edge"]["W2"])),
         _t4h(_pad_cols(encf["edge"]["b2"][None])),
         _bd4(_pad_rows(m3["e_top"]))],
        diff_halves=True)

    final_w = [fin["W1"], fin["b1"][None], fin["W2"], fin["b2"][None]]
    return _run_processing(
        N, NE, src3, dst3, prep_node3, prep_edge3, m3, final_w=final_w)
